# baseline (speedup 1.0000x reference)
import functools

import jax
import jax.numpy as jnp
from jax import lax
from jax.experimental import pallas as pl
from jax.experimental.pallas import tpu as pltpu
from jax.experimental.pallas import tpu_sc as plsc

NC = 2
NS = 16
NW = NC * NS
LANES = 16

B = 16384
F = 26
D = 16
CB = 128
ROWS = F * CB
CHUNKS = B // (NW * CB)


def _body(x_hbm, emb_hbm, bias_hbm, off_hbm, out_hbm, idx_v, rows_v, off_v, sem):
    wid = lax.axis_index("s") * NC + lax.axis_index("c")

    pltpu.sync_copy(off_hbm, off_v)

    for c in range(CHUNKS):
        pltpu.sync_copy(x_hbm.at[wid, c], idx_v)

        def add_off(j, _):
            for k in range(CB // LANES):
                s = pl.ds(k * LANES, LANES)
                idx_v[j, s] = idx_v[j, s] + off_v[j, s]
            return _

        lax.fori_loop(0, F, add_off, None)

        pltpu.sync_copy(bias_hbm, rows_v)

        def fire(j, _):
            pltpu.async_copy(emb_hbm.at[idx_v.at[j]], rows_v.at[j], sem, add=True)
            return _

        lax.fori_loop(0, F, fire, None)

        pltpu.make_async_copy(bias_hbm, rows_v, sem).wait()

        pltpu.sync_copy(rows_v, out_hbm.at[wid, c])


@jax.jit
def _tokenizer(x_r, embeddings, bias_tile, off_tile):
    run = pl.kernel(
        _body,
        out_type=jax.ShapeDtypeStruct((NW, CHUNKS, F, CB, D), jnp.float32),
        mesh=plsc.VectorSubcoreMesh(core_axis_name="c", subcore_axis_name="s"),
        scratch_types=[
            pltpu.VMEM((F, CB), jnp.int32),
            pltpu.VMEM((F, CB, D), jnp.float32),
            pltpu.VMEM((F, CB), jnp.int32),
            pltpu.SemaphoreType.DMA,
        ],
        compiler_params=pltpu.CompilerParams(use_tc_tiling_on_sc=False),
    )
    return run(x_r, embeddings, bias_tile, off_tile)


def kernel(x, embeddings, bias, category_offsets):
    x_r = x.reshape(NW, CHUNKS, F, CB)
    off_tile = jnp.tile(category_offsets, CB).reshape(F, CB)
    bias_tile = jnp.tile(bias, (CB, 1)).reshape(F, CB, D)
    out = _tokenizer(x_r, embeddings, bias_tile, off_tile)
    return out.reshape(B, F, D)
